# baseline (speedup 1.0000x reference)
import jax
import jax.numpy as jnp
from jax import lax
from jax.experimental import pallas as pl
from jax.experimental.pallas import tpu as pltpu
from jax.experimental.pallas import tpu_sc as plsc

B, S = 4096, 200
WORD_DIM = 128
POS_ROWS = 201
POS_SIZE = 16
OUT_DIM = WORD_DIM + 2 * POS_SIZE

NC, NS = 2, 16
NW = NC * NS
N = B * S
PER_W = N // NW
CHUNK = 256
NITER = PER_W // CHUNK
WSUB = CHUNK // 128


def _emb_kernel(widx_hbm, p1idx_hbm, p2idx_hbm, wtab_hbm, p1tab_hbm, p2tab_hbm,
                out_hbm, widx_v, p1idx_v, p2idx_v, outbuf_v, p1tab_v, p2tab_v,
                sem_w):
    wid = lax.axis_index("s") * NC + lax.axis_index("c")
    base = wid * PER_W

    pltpu.sync_copy(p1tab_hbm, p1tab_v)
    pltpu.sync_copy(p2tab_hbm, p2tab_v)

    lane = lax.iota(jnp.int32, 16)

    @pl.loop(0, NITER)
    def _(it):
        off = base + it * CHUNK
        pltpu.sync_copy(widx_hbm.at[pl.ds(off, CHUNK)], widx_v)
        pltpu.sync_copy(p1idx_hbm.at[pl.ds(off, CHUNK)], p1idx_v)
        pltpu.sync_copy(p2idx_hbm.at[pl.ds(off, CHUNK)], p2idx_v)

        copies = [
            pltpu.async_copy(
                wtab_hbm.at[widx_v.at[pl.ds(h * 128, 128)]],
                outbuf_v.at[pl.ds(h * 128, 128), pl.ds(0, WORD_DIM)],
                sem_w)
            for h in range(WSUB)
        ]

        @pl.loop(0, CHUNK // 16)
        def _(g):
            rowv = g * 16 + lane
            pv1 = p1idx_v[pl.ds(g * 16, 16)] * POS_SIZE
            pv2 = p2idx_v[pl.ds(g * 16, 16)] * POS_SIZE
            for c in range(POS_SIZE):
                v1 = plsc.load_gather(p1tab_v, [pv1 + c])
                plsc.store_scatter(
                    outbuf_v, [rowv, jnp.full((16,), WORD_DIM + c, jnp.int32)],
                    v1)
                v2 = plsc.load_gather(p2tab_v, [pv2 + c])
                plsc.store_scatter(
                    outbuf_v,
                    [rowv, jnp.full((16,), WORD_DIM + POS_SIZE + c, jnp.int32)],
                    v2)

        for cp in copies:
            cp.wait()
        pltpu.sync_copy(outbuf_v, out_hbm.at[pl.ds(off, CHUNK)])


@jax.jit
def _run(widx2d, p1idx, p2idx, word_table, pos1_flat, pos2_flat):
    mesh = plsc.VectorSubcoreMesh(core_axis_name="c", subcore_axis_name="s",
                                  num_cores=NC, num_subcores=NS)
    return pl.kernel(
        _emb_kernel,
        out_type=jax.ShapeDtypeStruct((N, OUT_DIM), jnp.float32),
        mesh=mesh,
        compiler_params=pltpu.CompilerParams(needs_layout_passes=False,
                                             use_tc_tiling_on_sc=True),
        scratch_types=[
            pltpu.VMEM((CHUNK,), jnp.int32),
            pltpu.VMEM((CHUNK,), jnp.int32),
            pltpu.VMEM((CHUNK,), jnp.int32),
            pltpu.VMEM((CHUNK, OUT_DIM), jnp.float32),
            pltpu.VMEM((POS_ROWS * POS_SIZE,), jnp.float32),
            pltpu.VMEM((POS_ROWS * POS_SIZE,), jnp.float32),
            pltpu.SemaphoreType.DMA,
        ],
    )(widx2d, p1idx, p2idx, word_table, pos1_flat, pos2_flat)


def kernel(input_word, input_pos1, input_pos2, word_table, pos1_table, pos2_table):
    widx2d = input_word.reshape(-1).astype(jnp.int32)
    p1idx = input_pos1.reshape(-1).astype(jnp.int32)
    p2idx = input_pos2.reshape(-1).astype(jnp.int32)
    out = _run(widx2d, p1idx, p2idx, word_table,
               pos1_table.reshape(-1), pos2_table.reshape(-1))
    return out.reshape(B, S, OUT_DIM)


# speedup vs baseline: 1.1906x; 1.1906x over previous
import jax
import jax.numpy as jnp
from jax import lax
from jax.experimental import pallas as pl
from jax.experimental.pallas import tpu as pltpu
from jax.experimental.pallas import tpu_sc as plsc

B, S = 4096, 200
WORD_DIM = 128
POS_ROWS = 201
POS_SIZE = 16
OUT_DIM = WORD_DIM + 2 * POS_SIZE

NC, NS = 2, 16
NW = NC * NS
N = B * S
PER_W = N // NW
CHUNK = 128
NITER = PER_W // CHUNK
WSUB = CHUNK // 128
NBUF = 2


def _emb_kernel(widx_hbm, p1idx_hbm, p2idx_hbm, wtab_hbm, p1tab_hbm, p2tab_hbm,
                out_hbm,
                widx_v0, p1idx_v0, p2idx_v0, outbuf_v0,
                widx_v1, p1idx_v1, p2idx_v1, outbuf_v1,
                p1tab_v, p2tab_v,
                sem_i0, sem_g0, sem_w0, sem_i1, sem_g1, sem_w1):
    wid = lax.axis_index("s") * NC + lax.axis_index("c")
    base = wid * PER_W

    bufs = [(widx_v0, p1idx_v0, p2idx_v0, outbuf_v0, sem_i0, sem_g0, sem_w0),
            (widx_v1, p1idx_v1, p2idx_v1, outbuf_v1, sem_i1, sem_g1, sem_w1)]

    def idx_copies(i, k):
        widx_v, p1idx_v, p2idx_v, _, sem_i, _, _ = bufs[k]
        off = base + i * CHUNK
        sl = pl.ds(off, CHUNK)
        return [pltpu.make_async_copy(widx_hbm.at[sl], widx_v, sem_i),
                pltpu.make_async_copy(p1idx_hbm.at[sl], p1idx_v, sem_i),
                pltpu.make_async_copy(p2idx_hbm.at[sl], p2idx_v, sem_i)]

    def gather_copies(k):
        widx_v, _, _, outbuf_v, _, sem_g, _ = bufs[k]
        return [
            pltpu.make_async_copy(
                wtab_hbm.at[widx_v.at[pl.ds(h * 128, 128)]],
                outbuf_v.at[pl.ds(h * 128, 128), pl.ds(0, WORD_DIM)],
                sem_g)
            for h in range(WSUB)
        ]

    def wb_copy(i, k):
        _, _, _, outbuf_v, _, _, sem_w = bufs[k]
        off = base + i * CHUNK
        return pltpu.make_async_copy(outbuf_v, out_hbm.at[pl.ds(off, CHUNK)],
                                     sem_w)

    pltpu.sync_copy(p1tab_hbm, p1tab_v)
    pltpu.sync_copy(p2tab_hbm, p2tab_v)

    lane = lax.iota(jnp.int32, 16)

    for c in idx_copies(0, 0):
        c.start()

    @pl.loop(0, NITER // NBUF)
    def _(g):
        for k in range(NBUF):
            i = g * NBUF + k
            _, p1idx_v, p2idx_v, outbuf_v, _, _, _ = bufs[k]

            for c in idx_copies(i, k):
                c.wait()

            @pl.when(g >= 1)
            def _():
                wb_copy(i - NBUF, k).wait()

            for c in gather_copies(k):
                c.start()

            @pl.loop(0, CHUNK // 16)
            def _(gr):
                rowv = gr * 16 + lane
                pv1 = p1idx_v[pl.ds(gr * 16, 16)] * POS_SIZE
                pv2 = p2idx_v[pl.ds(gr * 16, 16)] * POS_SIZE
                for c in range(POS_SIZE):
                    v1 = plsc.load_gather(p1tab_v, [pv1 + c])
                    plsc.store_scatter(
                        outbuf_v,
                        [rowv, jnp.full((16,), WORD_DIM + c, jnp.int32)], v1)
                    v2 = plsc.load_gather(p2tab_v, [pv2 + c])
                    plsc.store_scatter(
                        outbuf_v,
                        [rowv,
                         jnp.full((16,), WORD_DIM + POS_SIZE + c, jnp.int32)],
                        v2)

            for c in gather_copies(k):
                c.wait()

            @pl.when(i + 1 < NITER)
            def _():
                for c in idx_copies(i + 1, (k + 1) % NBUF):
                    c.start()

            wb_copy(i, k).start()

    for k in range(NBUF):
        wb_copy(NITER - NBUF + k, k).wait()


@jax.jit
def _run(widx, p1idx, p2idx, word_table, pos1_flat, pos2_flat):
    mesh = plsc.VectorSubcoreMesh(core_axis_name="c", subcore_axis_name="s",
                                  num_cores=NC, num_subcores=NS)
    return pl.kernel(
        _emb_kernel,
        out_type=jax.ShapeDtypeStruct((N, OUT_DIM), jnp.float32),
        mesh=mesh,
        compiler_params=pltpu.CompilerParams(needs_layout_passes=False,
                                             use_tc_tiling_on_sc=True),
        scratch_types=[
            pltpu.VMEM((CHUNK,), jnp.int32),
            pltpu.VMEM((CHUNK,), jnp.int32),
            pltpu.VMEM((CHUNK,), jnp.int32),
            pltpu.VMEM((CHUNK, OUT_DIM), jnp.float32),
            pltpu.VMEM((CHUNK,), jnp.int32),
            pltpu.VMEM((CHUNK,), jnp.int32),
            pltpu.VMEM((CHUNK,), jnp.int32),
            pltpu.VMEM((CHUNK, OUT_DIM), jnp.float32),
            pltpu.VMEM((POS_ROWS * POS_SIZE,), jnp.float32),
            pltpu.VMEM((POS_ROWS * POS_SIZE,), jnp.float32),
            pltpu.SemaphoreType.DMA,
            pltpu.SemaphoreType.DMA,
            pltpu.SemaphoreType.DMA,
            pltpu.SemaphoreType.DMA,
            pltpu.SemaphoreType.DMA,
            pltpu.SemaphoreType.DMA,
        ],
    )(widx, p1idx, p2idx, word_table, pos1_flat, pos2_flat)


def kernel(input_word, input_pos1, input_pos2, word_table, pos1_table, pos2_table):
    widx = input_word.reshape(-1).astype(jnp.int32)
    p1idx = input_pos1.reshape(-1).astype(jnp.int32)
    p2idx = input_pos2.reshape(-1).astype(jnp.int32)
    out = _run(widx, p1idx, p2idx, word_table,
               pos1_table.reshape(-1), pos2_table.reshape(-1))
    return out.reshape(B, S, OUT_DIM)


# speedup vs baseline: 1.3011x; 1.0928x over previous
import jax
import jax.numpy as jnp
from jax import lax
from jax.experimental import pallas as pl
from jax.experimental.pallas import tpu as pltpu
from jax.experimental.pallas import tpu_sc as plsc

B, S = 4096, 200
WORD_DIM = 128
POS_ROWS = 201
POS_SIZE = 16
OUT_DIM = WORD_DIM + 2 * POS_SIZE

NC, NS = 2, 16
NW = NC * NS
N = B * S
PER_W = N // NW
CHUNK = 128
NITER = PER_W // CHUNK
BLK = 10
IBLK = BLK * CHUNK
NBLK = NITER // BLK
NSB = NBLK // 2


def _emb_kernel(widx_hbm, p1idx_hbm, p2idx_hbm, wtab_hbm, p1tab_hbm, p2tab_hbm,
                out_hbm,
                widx_v0, p1idx_v0, p2idx_v0,
                widx_v1, p1idx_v1, p2idx_v1,
                outbuf_v0, outbuf_v1, p1tab_v, p2tab_v,
                sem_i0, sem_i1, sem_g0, sem_g1, sem_w0, sem_w1):
    wid = lax.axis_index("s") * NC + lax.axis_index("c")
    base = wid * PER_W

    idxbufs = [(widx_v0, p1idx_v0, p2idx_v0, sem_i0),
               (widx_v1, p1idx_v1, p2idx_v1, sem_i1)]
    outbufs = [(outbuf_v0, sem_g0, sem_w0), (outbuf_v1, sem_g1, sem_w1)]

    def idxblk_copies(b, kb):
        widx_v, p1idx_v, p2idx_v, sem_i = idxbufs[kb]
        sl = pl.ds(base + b * IBLK, IBLK)
        return [pltpu.make_async_copy(widx_hbm.at[sl], widx_v, sem_i),
                pltpu.make_async_copy(p1idx_hbm.at[sl], p1idx_v, sem_i),
                pltpu.make_async_copy(p2idx_hbm.at[sl], p2idx_v, sem_i)]

    def gather_copy(j, k, kb):
        widx_v = idxbufs[kb][0]
        outbuf_v, sem_g, _ = outbufs[k]
        return pltpu.make_async_copy(
            wtab_hbm.at[widx_v.at[pl.ds(j * CHUNK, CHUNK)]],
            outbuf_v.at[:, pl.ds(0, WORD_DIM)],
            sem_g)

    def wb_copy(i, k):
        outbuf_v, _, sem_w = outbufs[k]
        return pltpu.make_async_copy(
            outbuf_v, out_hbm.at[pl.ds(base + i * CHUNK, CHUNK)], sem_w)

    pltpu.sync_copy(p1tab_hbm, p1tab_v)
    pltpu.sync_copy(p2tab_hbm, p2tab_v)

    lane = lax.iota(jnp.int32, 16)

    for c in idxblk_copies(0, 0):
        c.start()
    for c in idxblk_copies(0, 0):
        c.wait()
    for c in idxblk_copies(1, 1):
        c.start()
    gather_copy(0, 0, 0).start()

    @pl.loop(0, NSB)
    def _(sb):
        for bi in range(2):
            for j in range(BLK):
                ci = bi * BLK + j
                k = ci % 2
                i = sb * (2 * BLK) + ci
                _, p1idx_v, p2idx_v, _ = idxbufs[bi]
                outbuf_v = outbufs[k][0]
                ioff = j * CHUNK

                @pl.loop(0, CHUNK // 16)
                def _(gr):
                    rowv = gr * 16 + lane
                    pv1 = p1idx_v[pl.ds(ioff + gr * 16, 16)] * POS_SIZE
                    pv2 = p2idx_v[pl.ds(ioff + gr * 16, 16)] * POS_SIZE
                    for c in range(POS_SIZE):
                        v1 = plsc.load_gather(p1tab_v, [pv1 + c])
                        plsc.store_scatter(
                            outbuf_v,
                            [rowv, jnp.full((16,), WORD_DIM + c, jnp.int32)],
                            v1)
                        v2 = plsc.load_gather(p2tab_v, [pv2 + c])
                        plsc.store_scatter(
                            outbuf_v,
                            [rowv,
                             jnp.full((16,), WORD_DIM + POS_SIZE + c,
                                      jnp.int32)],
                            v2)

                gather_copy(j, k, bi).wait()

                if ci == 0:
                    @pl.when(sb >= 1)
                    def _():
                        wb_copy(i - 1, 1 - k).wait()
                else:
                    wb_copy(i - 1, 1 - k).wait()

                if j == BLK - 1:
                    last = (bi == 1)

                    def _boundary(nb_bi=1 - bi, sb_=sb, bi_=bi):
                        for c in idxblk_copies(sb_ * 2 + bi_ + 1, nb_bi):
                            c.wait()

                        @pl.when(sb_ * 2 + bi_ + 2 < NBLK)
                        def _():
                            for c in idxblk_copies(sb_ * 2 + bi_ + 2, bi_):
                                c.start()

                        gather_copy(0, 1 - k, nb_bi).start()

                    if last:
                        @pl.when(sb < NSB - 1)
                        def _():
                            _boundary()
                    else:
                        _boundary()
                else:
                    gather_copy(j + 1, 1 - k, bi).start()

                wb_copy(i, k).start()

    wb_copy(NITER - 1, (NITER - 1) % 2).wait()


@jax.jit
def _run(widx, p1idx, p2idx, word_table, pos1_flat, pos2_flat):
    mesh = plsc.VectorSubcoreMesh(core_axis_name="c", subcore_axis_name="s",
                                  num_cores=NC, num_subcores=NS)
    return pl.kernel(
        _emb_kernel,
        out_type=jax.ShapeDtypeStruct((N, OUT_DIM), jnp.float32),
        mesh=mesh,
        compiler_params=pltpu.CompilerParams(needs_layout_passes=False,
                                             use_tc_tiling_on_sc=True),
        scratch_types=[
            pltpu.VMEM((IBLK,), jnp.int32),
            pltpu.VMEM((IBLK,), jnp.int32),
            pltpu.VMEM((IBLK,), jnp.int32),
            pltpu.VMEM((IBLK,), jnp.int32),
            pltpu.VMEM((IBLK,), jnp.int32),
            pltpu.VMEM((IBLK,), jnp.int32),
            pltpu.VMEM((CHUNK, OUT_DIM), jnp.float32),
            pltpu.VMEM((CHUNK, OUT_DIM), jnp.float32),
            pltpu.VMEM((POS_ROWS * POS_SIZE,), jnp.float32),
            pltpu.VMEM((POS_ROWS * POS_SIZE,), jnp.float32),
            pltpu.SemaphoreType.DMA,
            pltpu.SemaphoreType.DMA,
            pltpu.SemaphoreType.DMA,
            pltpu.SemaphoreType.DMA,
            pltpu.SemaphoreType.DMA,
            pltpu.SemaphoreType.DMA,
        ],
    )(widx, p1idx, p2idx, word_table, pos1_flat, pos2_flat)


def kernel(input_word, input_pos1, input_pos2, word_table, pos1_table, pos2_table):
    widx = input_word.reshape(-1).astype(jnp.int32)
    p1idx = input_pos1.reshape(-1).astype(jnp.int32)
    p2idx = input_pos2.reshape(-1).astype(jnp.int32)
    out = _run(widx, p1idx, p2idx, word_table,
               pos1_table.reshape(-1), pos2_table.reshape(-1))
    return out.reshape(B, S, OUT_DIM)
